# baseline (speedup 1.0000x reference)
import functools
from math import sqrt

import jax
import jax.numpy as jnp
from jax import lax
from jax.experimental import pallas as pl
from jax.experimental.pallas import tpu as pltpu
from jax.experimental.pallas import tpu_sc as plsc

N = 10000
E = 160000
D = 256
DH = 128
NC = 2
NS = 16
LANES = 16
CH = 128
NCHT = E // CH
MAXCH = -(-NCHT // NS)
ROWS_MAIN = 624
ROWS_EXTRA = 8
EPOCHS = 5
LR = 0.01
B1, B2, EPS = 0.9, 0.999, 1e-8
ER = E // DH
_F32 = jnp.float32


def _take16(v, idx):
    return lax.gather(
        v, idx[:, None],
        dimension_numbers=lax.GatherDimensionNumbers(
            offset_dims=(), collapsed_slice_dims=(0,),
            start_index_map=(0,)),
        slice_sizes=(1,),
        mode=lax.GatherScatterMode.PROMISE_IN_BOUNDS)


def _mm_body(x_ref, w_ref, o_ref):
    o_ref[...] = jnp.dot(x_ref[...], w_ref[...],
                         preferred_element_type=_F32)


_BN = 1000
_matmul = pl.pallas_call(
    _mm_body,
    grid=(NC, N // _BN),
    in_specs=[
        pl.BlockSpec((_BN, D), lambda c, i: (i, 0)),
        pl.BlockSpec((D, DH), lambda c, i: (0, c)),
    ],
    out_specs=pl.BlockSpec((_BN, DH), lambda c, i: (c * (N // _BN) + i, 0)),
    out_shape=jax.ShapeDtypeStruct((NC * N, DH), _F32),
)


_mesh = plsc.VectorSubcoreMesh(core_axis_name="c", subcore_axis_name="s")


@functools.partial(
    pl.kernel,
    out_type=jax.ShapeDtypeStruct((NC * N, DH), _F32),
    mesh=_mesh,
    scratch_types=(
        [pltpu.VMEM_SHARED((N, DH), _F32)]
        + [pltpu.VMEM((CH,), jnp.int32) for _ in range(2)]
        + [pltpu.VMEM((CH,), jnp.int32) for _ in range(2)]
        + [pltpu.VMEM((CH,), _F32) for _ in range(2)]
        + [pltpu.VMEM((CH, DH), _F32) for _ in range(2)]
        + [pltpu.SemaphoreType.DMA] * 4
    ),
)
def _fwd(h_hbm, src_hbm, dst_hbm, srcp_hbm, dstp_hbm, em_hbm, pm_hbm,
         diff_hbm, acc_sh, ix0, ix1, dx0, dx1, sc0, sc1, rw0, rw1,
         g0, g1, x0, x1):
    IX = (ix0, ix1)
    DX = (dx0, dx1)
    SC = (sc0, sc1)
    RW = (rw0, rw1)
    G = (g0, g1)
    X = (x0, x1)
    c = lax.axis_index("c")
    s = lax.axis_index("s")
    cN = c * N

    def _zrow(e, _):
        for j in range(DH // LANES):
            rw0[e, pl.ds(j * LANES, LANES)] = jnp.zeros((LANES,), _F32)
        return 0

    lax.fori_loop(0, CH, _zrow, 0)
    base_r = pl.multiple_of(s * ROWS_MAIN, 8)
    for q in range(ROWS_MAIN // CH):
        pltpu.sync_copy(rw0.at[pl.ds(0, CH)],
                        acc_sh.at[pl.ds(base_r + q * CH, CH)])
    _rem = ROWS_MAIN % CH
    pltpu.sync_copy(
        rw0.at[pl.ds(0, _rem)],
        acc_sh.at[pl.ds(base_r + (ROWS_MAIN // CH) * CH, _rem)])

    @pl.when(s < 2)
    def _():
        eb = pl.multiple_of(NS * ROWS_MAIN + s * ROWS_EXTRA, 8)
        pltpu.sync_copy(rw0.at[pl.ds(0, ROWS_EXTRA)],
                        acc_sh.at[pl.ds(eb, ROWS_EXTRA)])

    plsc.subcore_barrier()

    for sref, dref, mref, sign in (
        (src_hbm, dst_hbm, em_hbm, 1.0),
        (srcp_hbm, dstp_hbm, pm_hbm, -1.0),
    ):
        def _prep(i, sl, sref=sref, dref=dref, mref=mref, sign=sign):
            @pl.when(jnp.asarray(i >= 2))
            def _():
                pltpu.make_async_copy(RW[sl], acc_sh.at[DX[sl]],
                                      X[sl]).wait()

            ci = s + i * NS
            base = pl.multiple_of(ci * CH, 8)
            pltpu.sync_copy(sref.at[pl.ds(base, CH)], IX[sl])
            pltpu.sync_copy(dref.at[pl.ds(base, CH)], DX[sl])
            pltpu.sync_copy(mref.at[pl.ds(base, CH)], SC[sl])

            def _vec(k, _):
                o = pl.multiple_of(k * LANES, 8)
                mv = SC[sl][pl.ds(o, LANES)]
                SC[sl][pl.ds(o, LANES)] = sign / (1.0 + jnp.exp(-mv))
                iv = IX[sl][pl.ds(o, LANES)]
                IX[sl][pl.ds(o, LANES)] = iv + cN
                return 0

            lax.fori_loop(0, CH // LANES, _vec, 0)
            pltpu.async_copy(h_hbm.at[IX[sl]], RW[sl], G[sl])

        def _proc(sl):
            pltpu.make_async_copy(h_hbm.at[IX[sl]], RW[sl], G[sl]).wait()

            def _scale(k, _):
                o = pl.multiple_of(k * LANES, 8)
                sv = SC[sl][pl.ds(o, LANES)]
                for i2 in range(LANES):
                    sc = sv[i2]
                    for j in range(DH // LANES):
                        RW[sl][o + i2, pl.ds(j * LANES, LANES)] = (
                            RW[sl][o + i2, pl.ds(j * LANES, LANES)] * sc)
                return 0

            lax.fori_loop(0, CH // LANES, _scale, 0)
            pltpu.async_copy(RW[sl], acc_sh.at[DX[sl]], X[sl], add=True)

        _prep(0, 0)

        def _body(u, _):
            for half in range(2):
                i = 2 * u + half

                @pl.when(s + (i + 1) * NS < NCHT)
                def _():
                    _prep(i + 1, 1 - half)

                _proc(half)
            return 0

        lax.fori_loop(0, (MAXCH - 1) // 2, _body, 0)

        @pl.when(s + (MAXCH - 1) * NS < NCHT)
        def _():
            _proc((MAXCH - 1) % 2)

        for sl in range(2):
            pltpu.make_async_copy(RW[sl], acc_sh.at[DX[sl]],
                                  X[sl]).wait()

    plsc.subcore_barrier()

    for q in range(ROWS_MAIN // CH):
        pltpu.sync_copy(acc_sh.at[pl.ds(base_r + q * CH, CH)],
                        diff_hbm.at[pl.ds(cN + base_r + q * CH, CH)])
    pltpu.sync_copy(
        acc_sh.at[pl.ds(base_r + (ROWS_MAIN // CH) * CH, _rem)],
        diff_hbm.at[pl.ds(cN + base_r + (ROWS_MAIN // CH) * CH, _rem)])

    @pl.when(s < 2)
    def _():
        eb = pl.multiple_of(NS * ROWS_MAIN + s * ROWS_EXTRA, 8)
        pltpu.sync_copy(acc_sh.at[pl.ds(eb, ROWS_EXTRA)],
                        diff_hbm.at[pl.ds(cN + eb, ROWS_EXTRA)])


@functools.partial(
    pl.kernel,
    out_type=jax.ShapeDtypeStruct((NC * 2 * E,), _F32),
    mesh=_mesh,
    scratch_types=(
        [pltpu.VMEM((CH,), jnp.int32) for _ in range(2)]
        + [pltpu.VMEM((CH,), jnp.int32) for _ in range(2)]
        + [pltpu.VMEM((CH,), _F32)]
        + [pltpu.VMEM((CH, DH), _F32) for _ in range(2)]
        + [pltpu.VMEM((CH, DH), _F32) for _ in range(2)]
        + [pltpu.SemaphoreType.DMA] * 2
    ),
)
def _bwd(h_hbm, diff_hbm, src_hbm, dst_hbm, srcp_hbm, dstp_hbm,
         dots_hbm, ix0, ix1, dx0, dx1, dots_v, hr0, hr1, dr0, dr1,
         g0, g1):
    IX = (ix0, ix1)
    DX = (dx0, dx1)
    HR = (hr0, hr1)
    DR = (dr0, dr1)
    G = (g0, g1)
    c = lax.axis_index("c")
    s = lax.axis_index("s")
    cN = c * N
    lane = lax.iota(jnp.int32, LANES)

    for g, (sref, dref) in enumerate(((src_hbm, dst_hbm),
                                      (srcp_hbm, dstp_hbm))):
        def _prep(i, sl, sref=sref, dref=dref):
            ci = s + i * NS
            base = pl.multiple_of(ci * CH, 8)
            pltpu.sync_copy(sref.at[pl.ds(base, CH)], IX[sl])
            pltpu.sync_copy(dref.at[pl.ds(base, CH)], DX[sl])

            def _vec(k, _):
                o = pl.multiple_of(k * LANES, 8)
                IX[sl][pl.ds(o, LANES)] = IX[sl][pl.ds(o, LANES)] + cN
                DX[sl][pl.ds(o, LANES)] = DX[sl][pl.ds(o, LANES)] + cN
                return 0

            lax.fori_loop(0, CH // LANES, _vec, 0)
            pltpu.async_copy(h_hbm.at[IX[sl]], HR[sl], G[sl])
            pltpu.async_copy(diff_hbm.at[DX[sl]], DR[sl], G[sl])

        def _proc(i, sl, g=g):
            pltpu.make_async_copy(h_hbm.at[IX[sl]], HR[sl], G[sl]).wait()
            pltpu.make_async_copy(diff_hbm.at[DX[sl]], DR[sl],
                                  G[sl]).wait()

            def _dot(k, _):
                o = pl.multiple_of(k * LANES, 8)
                tot = jnp.zeros((LANES,), _F32)
                for i2 in range(LANES):
                    e = o + i2
                    p = [HR[sl][e, pl.ds(j * LANES, LANES)] *
                         DR[sl][e, pl.ds(j * LANES, LANES)]
                         for j in range(DH // LANES)]
                    acc = (((p[0] + p[1]) + (p[2] + p[3]))
                           + ((p[4] + p[5]) + (p[6] + p[7])))
                    for sh in (1, 2, 4, 8):
                        acc = acc + _take16(acc, lane ^ sh)
                    tot = jnp.where(lane == i2, acc, tot)
                dots_v[pl.ds(o, LANES)] = tot
                return 0

            lax.fori_loop(0, CH // LANES, _dot, 0)
            ci = s + i * NS
            fb = pl.multiple_of((c * 2 + g) * E + ci * CH, 8)
            pltpu.sync_copy(dots_v, dots_hbm.at[pl.ds(fb, CH)])

        _prep(0, 0)

        def _body(u, _):
            for half in range(2):
                i = 2 * u + half

                @pl.when(s + (i + 1) * NS < NCHT)
                def _():
                    _prep(i + 1, 1 - half)

                _proc(i, half)
            return 0

        lax.fori_loop(0, (MAXCH - 1) // 2, _body, 0)

        @pl.when(s + (MAXCH - 1) * NS < NCHT)
        def _():
            _proc(MAXCH - 1, (MAXCH - 1) % 2)


def _adam_body(t, diff_ref, dots_ref, em_ref, pm_ref, mem_ref, mpm_ref,
               vem_ref, vpm_ref, oem_ref, opm_ref, omem_ref, ompm_ref,
               ovem_ref, ovpm_ref):
    inv_l = lax.rsqrt(jnp.sum(diff_ref[...] * diff_ref[...]))
    em = em_ref[...]
    pm = pm_ref[...]
    se = jax.nn.sigmoid(em)
    sp = jax.nn.sigmoid(pm)
    d_em = dots_ref[0:ER] + dots_ref[2 * ER:3 * ER]
    d_pm = dots_ref[ER:2 * ER] + dots_ref[3 * ER:4 * ER]
    g_em = se * (1.0 - se) * d_em * inv_l
    g_pm = -sp * (1.0 - sp) * d_pm * inv_l
    bc1 = 1.0 - B1 ** t
    bc2 = 1.0 - B2 ** t
    m_em = B1 * mem_ref[...] + (1.0 - B1) * g_em
    m_pm = B1 * mpm_ref[...] + (1.0 - B1) * g_pm
    v_em = B2 * vem_ref[...] + (1.0 - B2) * g_em * g_em
    v_pm = B2 * vpm_ref[...] + (1.0 - B2) * g_pm * g_pm
    oem_ref[...] = em - LR * (m_em / bc1) / (jnp.sqrt(v_em / bc2) + EPS)
    opm_ref[...] = pm - LR * (m_pm / bc1) / (jnp.sqrt(v_pm / bc2) + EPS)
    omem_ref[...] = m_em
    ompm_ref[...] = m_pm
    ovem_ref[...] = v_em
    ovpm_ref[...] = v_pm


_adam_calls = [
    pl.pallas_call(
        functools.partial(_adam_body, t),
        out_shape=[jax.ShapeDtypeStruct((ER, DH), _F32)] * 6,
    )
    for t in range(1, EPOCHS + 1)
]


def kernel(x, edge_index, perturbed_edge_index, W):
    std = sqrt(2.0) * sqrt(2.0 / (2.0 * N))
    ka, kb = jax.random.split(jax.random.key(1))
    em = jax.random.normal(ka, (E,), _F32) * std
    pm = jax.random.normal(kb, (E,), _F32) * std

    src, dst = edge_index[0], edge_index[1]
    srcp, dstp = perturbed_edge_index[0], perturbed_edge_index[1]

    h = _matmul(x, W)

    m_em = jnp.zeros((ER, DH), _F32)
    m_pm = jnp.zeros((ER, DH), _F32)
    v_em = jnp.zeros((ER, DH), _F32)
    v_pm = jnp.zeros((ER, DH), _F32)
    for t in range(1, EPOCHS + 1):
        diff = _fwd(h, src, dst, srcp, dstp, em, pm)
        dots = _bwd(h, diff, src, dst, srcp, dstp)
        em2, pm2, m_em, m_pm, v_em, v_pm = _adam_calls[t - 1](
            diff, dots.reshape(NC * 2 * ER, DH),
            em.reshape(ER, DH), pm.reshape(ER, DH),
            m_em, m_pm, v_em, v_pm)
        em = em2.reshape(E)
        pm = pm2.reshape(E)
    return em, pm


# speedup vs baseline: 1.1970x; 1.1970x over previous
import functools
from math import sqrt

import jax
import jax.numpy as jnp
from jax import lax
from jax.experimental import pallas as pl
from jax.experimental.pallas import tpu as pltpu
from jax.experimental.pallas import tpu_sc as plsc

N = 10000
E = 160000
D = 256
DH = 128
NC = 2
NS = 16
LANES = 16
CH = 128
NCHT = E // CH
MAXCH = -(-NCHT // NS)
ROWS_MAIN = 624
ROWS_EXTRA = 8
EPOCHS = 5
LR = 0.01
B1, B2, EPS = 0.9, 0.999, 1e-8
ER = E // DH
_F32 = jnp.float32


def _take16(v, idx):
    return lax.gather(
        v, idx[:, None],
        dimension_numbers=lax.GatherDimensionNumbers(
            offset_dims=(), collapsed_slice_dims=(0,),
            start_index_map=(0,)),
        slice_sizes=(1,),
        mode=lax.GatherScatterMode.PROMISE_IN_BOUNDS)


def _mm_body(x_ref, w_ref, o_ref):
    o_ref[...] = jnp.dot(x_ref[...], w_ref[...],
                         preferred_element_type=_F32)


_BN = 1000
_matmul = pl.pallas_call(
    _mm_body,
    grid=(NC, N // _BN),
    in_specs=[
        pl.BlockSpec((_BN, D), lambda c, i: (i, 0)),
        pl.BlockSpec((D, DH), lambda c, i: (0, c)),
    ],
    out_specs=pl.BlockSpec((_BN, DH), lambda c, i: (c * (N // _BN) + i, 0)),
    out_shape=jax.ShapeDtypeStruct((NC * N, DH), _F32),
)


_mesh = plsc.VectorSubcoreMesh(core_axis_name="c", subcore_axis_name="s")


@functools.partial(
    pl.kernel,
    out_type=jax.ShapeDtypeStruct((NC * N, DH), _F32),
    mesh=_mesh,
    scratch_types=(
        [pltpu.VMEM_SHARED((N, DH), _F32)]
        + [pltpu.VMEM((CH,), jnp.int32) for _ in range(2)]
        + [pltpu.VMEM((CH,), jnp.int32) for _ in range(2)]
        + [pltpu.VMEM((CH,), _F32) for _ in range(2)]
        + [pltpu.VMEM((CH, DH), _F32) for _ in range(2)]
        + [pltpu.SemaphoreType.DMA] * 4
    ),
)
def _fwd(h_hbm, src_hbm, dst_hbm, srcp_hbm, dstp_hbm, em_hbm, pm_hbm,
         diff_hbm, acc_sh, ix0, ix1, dx0, dx1, sc0, sc1, rw0, rw1,
         g0, g1, x0, x1):
    IX = (ix0, ix1)
    DX = (dx0, dx1)
    SC = (sc0, sc1)
    RW = (rw0, rw1)
    G = (g0, g1)
    X = (x0, x1)
    c = lax.axis_index("c")
    s = lax.axis_index("s")
    cN = c * N

    def _zrow(e, _):
        for j in range(DH // LANES):
            rw0[e, pl.ds(j * LANES, LANES)] = jnp.zeros((LANES,), _F32)
        return 0

    lax.fori_loop(0, CH, _zrow, 0)
    base_r = pl.multiple_of(s * ROWS_MAIN, 8)
    for q in range(ROWS_MAIN // CH):
        pltpu.sync_copy(rw0.at[pl.ds(0, CH)],
                        acc_sh.at[pl.ds(base_r + q * CH, CH)])
    _rem = ROWS_MAIN % CH
    pltpu.sync_copy(
        rw0.at[pl.ds(0, _rem)],
        acc_sh.at[pl.ds(base_r + (ROWS_MAIN // CH) * CH, _rem)])

    @pl.when(s < 2)
    def _():
        eb = pl.multiple_of(NS * ROWS_MAIN + s * ROWS_EXTRA, 8)
        pltpu.sync_copy(rw0.at[pl.ds(0, ROWS_EXTRA)],
                        acc_sh.at[pl.ds(eb, ROWS_EXTRA)])

    plsc.subcore_barrier()

    for sref, dref, mref, sign in (
        (src_hbm, dst_hbm, em_hbm, 1.0),
        (srcp_hbm, dstp_hbm, pm_hbm, -1.0),
    ):
        def _prep(i, sl, sref=sref, dref=dref, mref=mref, sign=sign):
            @pl.when(jnp.asarray(i >= 2))
            def _():
                pltpu.make_async_copy(RW[sl], acc_sh.at[DX[sl]],
                                      X[sl]).wait()

            ci = s + i * NS
            base = pl.multiple_of(ci * CH, 8)
            pltpu.sync_copy(sref.at[pl.ds(base, CH)], IX[sl])
            pltpu.sync_copy(dref.at[pl.ds(base, CH)], DX[sl])
            pltpu.sync_copy(mref.at[pl.ds(base, CH)], SC[sl])

            def _vec(k, _):
                o = pl.multiple_of(k * LANES, 8)
                mv = SC[sl][pl.ds(o, LANES)]
                SC[sl][pl.ds(o, LANES)] = sign / (1.0 + jnp.exp(-mv))
                iv = IX[sl][pl.ds(o, LANES)]
                IX[sl][pl.ds(o, LANES)] = iv + cN
                return 0

            lax.fori_loop(0, CH // LANES, _vec, 0)
            pltpu.async_copy(h_hbm.at[IX[sl]], RW[sl], G[sl])

        def _proc(sl):
            pltpu.make_async_copy(h_hbm.at[IX[sl]], RW[sl], G[sl]).wait()

            def _scale(k, _):
                o = pl.multiple_of(k * LANES, 8)
                sv = SC[sl][pl.ds(o, LANES)]
                for i2 in range(LANES):
                    sc = sv[i2]
                    for j in range(DH // LANES):
                        RW[sl][o + i2, pl.ds(j * LANES, LANES)] = (
                            RW[sl][o + i2, pl.ds(j * LANES, LANES)] * sc)
                return 0

            lax.fori_loop(0, CH // LANES, _scale, 0)
            pltpu.async_copy(RW[sl], acc_sh.at[DX[sl]], X[sl], add=True)

        _prep(0, 0)

        def _body(u, _):
            for half in range(2):
                i = 2 * u + half

                @pl.when(s + (i + 1) * NS < NCHT)
                def _():
                    _prep(i + 1, 1 - half)

                _proc(half)
            return 0

        lax.fori_loop(0, (MAXCH - 1) // 2, _body, 0)

        @pl.when(s + (MAXCH - 1) * NS < NCHT)
        def _():
            _proc((MAXCH - 1) % 2)

        for sl in range(2):
            pltpu.make_async_copy(RW[sl], acc_sh.at[DX[sl]],
                                  X[sl]).wait()

    plsc.subcore_barrier()

    for q in range(ROWS_MAIN // CH):
        pltpu.sync_copy(acc_sh.at[pl.ds(base_r + q * CH, CH)],
                        diff_hbm.at[pl.ds(cN + base_r + q * CH, CH)])
    pltpu.sync_copy(
        acc_sh.at[pl.ds(base_r + (ROWS_MAIN // CH) * CH, _rem)],
        diff_hbm.at[pl.ds(cN + base_r + (ROWS_MAIN // CH) * CH, _rem)])

    @pl.when(s < 2)
    def _():
        eb = pl.multiple_of(NS * ROWS_MAIN + s * ROWS_EXTRA, 8)
        pltpu.sync_copy(acc_sh.at[pl.ds(eb, ROWS_EXTRA)],
                        diff_hbm.at[pl.ds(cN + eb, ROWS_EXTRA)])


@functools.partial(
    pl.kernel,
    out_type=jax.ShapeDtypeStruct((NC * 2 * E,), _F32),
    mesh=_mesh,
    scratch_types=(
        [pltpu.VMEM((CH,), jnp.int32) for _ in range(2)]
        + [pltpu.VMEM((CH,), jnp.int32) for _ in range(2)]
        + [pltpu.VMEM((CH,), _F32)]
        + [pltpu.VMEM((CH, DH), _F32) for _ in range(2)]
        + [pltpu.VMEM((CH, DH), _F32) for _ in range(2)]
        + [pltpu.SemaphoreType.DMA] * 2
    ),
)
def _bwd(h_hbm, diff_hbm, src_hbm, dst_hbm, srcp_hbm, dstp_hbm,
         dots_hbm, ix0, ix1, dx0, dx1, dots_v, hr0, hr1, dr0, dr1,
         g0, g1):
    IX = (ix0, ix1)
    DX = (dx0, dx1)
    HR = (hr0, hr1)
    DR = (dr0, dr1)
    G = (g0, g1)
    c = lax.axis_index("c")
    s = lax.axis_index("s")
    cN = c * N
    lane = lax.iota(jnp.int32, LANES)

    for g, (sref, dref) in enumerate(((src_hbm, dst_hbm),
                                      (srcp_hbm, dstp_hbm))):
        def _prep(i, sl, sref=sref, dref=dref):
            ci = s + i * NS
            base = pl.multiple_of(ci * CH, 8)
            pltpu.sync_copy(sref.at[pl.ds(base, CH)], IX[sl])
            pltpu.sync_copy(dref.at[pl.ds(base, CH)], DX[sl])

            def _vec(k, _):
                o = pl.multiple_of(k * LANES, 8)
                IX[sl][pl.ds(o, LANES)] = IX[sl][pl.ds(o, LANES)] + cN
                DX[sl][pl.ds(o, LANES)] = DX[sl][pl.ds(o, LANES)] + cN
                return 0

            lax.fori_loop(0, CH // LANES, _vec, 0)
            pltpu.async_copy(h_hbm.at[IX[sl]], HR[sl], G[sl])
            pltpu.async_copy(diff_hbm.at[DX[sl]], DR[sl], G[sl])

        def _proc(i, sl, g=g):
            pltpu.make_async_copy(h_hbm.at[IX[sl]], HR[sl], G[sl]).wait()
            pltpu.make_async_copy(diff_hbm.at[DX[sl]], DR[sl],
                                  G[sl]).wait()

            def _dot(k, _):
                o = pl.multiple_of(k * LANES, 8)
                tot = jnp.zeros((LANES,), _F32)
                for i2 in range(LANES):
                    e = o + i2
                    acc = (HR[sl][e, pl.ds(0, LANES)] *
                           DR[sl][e, pl.ds(0, LANES)])
                    for j in range(1, DH // LANES):
                        acc = acc + (
                            HR[sl][e, pl.ds(j * LANES, LANES)] *
                            DR[sl][e, pl.ds(j * LANES, LANES)])
                    for sh in (1, 2, 4, 8):
                        acc = acc + _take16(acc, lane ^ sh)
                    tot = jnp.where(lane == i2, acc, tot)
                dots_v[pl.ds(o, LANES)] = tot
                return 0

            lax.fori_loop(0, CH // LANES, _dot, 0)
            ci = s + i * NS
            fb = pl.multiple_of((c * 2 + g) * E + ci * CH, 8)
            pltpu.sync_copy(dots_v, dots_hbm.at[pl.ds(fb, CH)])

        _prep(0, 0)

        def _body(u, _):
            for half in range(2):
                i = 2 * u + half

                @pl.when(s + (i + 1) * NS < NCHT)
                def _():
                    _prep(i + 1, 1 - half)

                _proc(i, half)
            return 0

        lax.fori_loop(0, (MAXCH - 1) // 2, _body, 0)

        @pl.when(s + (MAXCH - 1) * NS < NCHT)
        def _():
            _proc(MAXCH - 1, (MAXCH - 1) % 2)


def _adam_body(t, diff_ref, dots_ref, em_ref, pm_ref, mem_ref, mpm_ref,
               vem_ref, vpm_ref, oem_ref, opm_ref, omem_ref, ompm_ref,
               ovem_ref, ovpm_ref):
    inv_l = lax.rsqrt(jnp.sum(diff_ref[...] * diff_ref[...]))
    em = em_ref[...]
    pm = pm_ref[...]
    se = jax.nn.sigmoid(em)
    sp = jax.nn.sigmoid(pm)
    d_em = dots_ref[0:ER] + dots_ref[2 * ER:3 * ER]
    d_pm = dots_ref[ER:2 * ER] + dots_ref[3 * ER:4 * ER]
    g_em = se * (1.0 - se) * d_em * inv_l
    g_pm = -sp * (1.0 - sp) * d_pm * inv_l
    bc1 = 1.0 - B1 ** t
    bc2 = 1.0 - B2 ** t
    m_em = B1 * mem_ref[...] + (1.0 - B1) * g_em
    m_pm = B1 * mpm_ref[...] + (1.0 - B1) * g_pm
    v_em = B2 * vem_ref[...] + (1.0 - B2) * g_em * g_em
    v_pm = B2 * vpm_ref[...] + (1.0 - B2) * g_pm * g_pm
    oem_ref[...] = em - LR * (m_em / bc1) / (jnp.sqrt(v_em / bc2) + EPS)
    opm_ref[...] = pm - LR * (m_pm / bc1) / (jnp.sqrt(v_pm / bc2) + EPS)
    omem_ref[...] = m_em
    ompm_ref[...] = m_pm
    ovem_ref[...] = v_em
    ovpm_ref[...] = v_pm


_adam_calls = [
    pl.pallas_call(
        functools.partial(_adam_body, t),
        out_shape=[jax.ShapeDtypeStruct((ER, DH), _F32)] * 6,
    )
    for t in range(1, EPOCHS + 1)
]


def kernel(x, edge_index, perturbed_edge_index, W):
    std = sqrt(2.0) * sqrt(2.0 / (2.0 * N))
    ka, kb = jax.random.split(jax.random.key(1))
    em = jax.random.normal(ka, (E,), _F32) * std
    pm = jax.random.normal(kb, (E,), _F32) * std

    src, dst = edge_index[0], edge_index[1]
    srcp, dstp = perturbed_edge_index[0], perturbed_edge_index[1]

    h = _matmul(x, W)

    m_em = jnp.zeros((ER, DH), _F32)
    m_pm = jnp.zeros((ER, DH), _F32)
    v_em = jnp.zeros((ER, DH), _F32)
    v_pm = jnp.zeros((ER, DH), _F32)
    for t in range(1, EPOCHS + 1):
        diff = _fwd(h, src, dst, srcp, dstp, em, pm)
        dots = _bwd(h, diff, src, dst, srcp, dstp)
        em2, pm2, m_em, m_pm, v_em, v_pm = _adam_calls[t - 1](
            diff, dots.reshape(NC * 2 * ER, DH),
            em.reshape(ER, DH), pm.reshape(ER, DH),
            m_em, m_pm, v_em, v_pm)
        em = em2.reshape(E)
        pm = pm2.reshape(E)
    return em, pm


# speedup vs baseline: 1.3625x; 1.1383x over previous
import functools
from math import sqrt

import jax
import jax.numpy as jnp
from jax import lax
from jax.experimental import pallas as pl
from jax.experimental.pallas import tpu as pltpu
from jax.experimental.pallas import tpu_sc as plsc

N = 10000
E = 160000
D = 256
DH = 128
NC = 2
NS = 16
LANES = 16
CH = 128
NCHT = E // CH
MAXCH = -(-NCHT // NS)
ROWS_MAIN = 624
ROWS_EXTRA = 8
EPOCHS = 5
LR = 0.01
B1, B2, EPS = 0.9, 0.999, 1e-8
ER = E // DH
_F32 = jnp.float32


def _take16(v, idx):
    return lax.gather(
        v, idx[:, None],
        dimension_numbers=lax.GatherDimensionNumbers(
            offset_dims=(), collapsed_slice_dims=(0,),
            start_index_map=(0,)),
        slice_sizes=(1,),
        mode=lax.GatherScatterMode.PROMISE_IN_BOUNDS)


def _mm_body(x_ref, w_ref, o_ref):
    o_ref[...] = jnp.dot(x_ref[...], w_ref[...],
                         preferred_element_type=_F32)


_BN = 1000
_matmul = pl.pallas_call(
    _mm_body,
    grid=(NC, N // _BN),
    in_specs=[
        pl.BlockSpec((_BN, D), lambda c, i: (i, 0)),
        pl.BlockSpec((D, DH), lambda c, i: (0, c)),
    ],
    out_specs=pl.BlockSpec((_BN, DH), lambda c, i: (c * (N // _BN) + i, 0)),
    out_shape=jax.ShapeDtypeStruct((NC * N, DH), _F32),
)


_mesh = plsc.VectorSubcoreMesh(core_axis_name="c", subcore_axis_name="s")


@functools.partial(
    pl.kernel,
    out_type=jax.ShapeDtypeStruct((NC * N, DH), _F32),
    mesh=_mesh,
    scratch_types=(
        [pltpu.VMEM_SHARED((N, DH), _F32)]
        + [pltpu.VMEM((2, CH), jnp.int32) for _ in range(2)]
        + [pltpu.VMEM((CH,), jnp.int32) for _ in range(2)]
        + [pltpu.VMEM((CH,), jnp.int32) for _ in range(2)]
        + [pltpu.VMEM((CH,), _F32) for _ in range(2)]
        + [pltpu.VMEM((CH, DH), _F32) for _ in range(2)]
        + [pltpu.SemaphoreType.DMA] * 4
    ),
)
def _fwd(h_hbm, fpk_hbm, em_hbm, pm_hbm,
         diff_hbm, acc_sh, pk0, pk1, ix0, ix1, dx0, dx1, sc0, sc1,
         rw0, rw1, g0, g1, x0, x1):
    PK = (pk0, pk1)
    IX = (ix0, ix1)
    DX = (dx0, dx1)
    SC = (sc0, sc1)
    RW = (rw0, rw1)
    G = (g0, g1)
    X = (x0, x1)
    c = lax.axis_index("c")
    s = lax.axis_index("s")
    cN = c * N

    def _zrow(e, _):
        for j in range(DH // LANES):
            rw0[e, pl.ds(j * LANES, LANES)] = jnp.zeros((LANES,), _F32)
        return 0

    lax.fori_loop(0, CH, _zrow, 0)
    base_r = pl.multiple_of(s * ROWS_MAIN, 8)
    for q in range(ROWS_MAIN // CH):
        pltpu.sync_copy(rw0.at[pl.ds(0, CH)],
                        acc_sh.at[pl.ds(base_r + q * CH, CH)])
    _rem = ROWS_MAIN % CH
    pltpu.sync_copy(
        rw0.at[pl.ds(0, _rem)],
        acc_sh.at[pl.ds(base_r + (ROWS_MAIN // CH) * CH, _rem)])

    @pl.when(s < 2)
    def _():
        eb = pl.multiple_of(NS * ROWS_MAIN + s * ROWS_EXTRA, 8)
        pltpu.sync_copy(rw0.at[pl.ds(0, ROWS_EXTRA)],
                        acc_sh.at[pl.ds(eb, ROWS_EXTRA)])

    plsc.subcore_barrier()

    for g, (mref, sign) in enumerate(((em_hbm, 1.0), (pm_hbm, -1.0))):
        def _prep(i, sl, g=g, mref=mref, sign=sign):
            @pl.when(jnp.asarray(i >= 2))
            def _():
                pltpu.make_async_copy(RW[sl], acc_sh.at[DX[sl]],
                                      X[sl]).wait()

            ci = s + i * NS
            base = pl.multiple_of(ci * CH, 8)
            pltpu.sync_copy(fpk_hbm.at[c, g, ci], PK[sl])
            pltpu.sync_copy(mref.at[pl.ds(base, CH)], SC[sl])

            def _vec(k, _):
                o = pl.multiple_of(k * LANES, 8)
                mv = SC[sl][pl.ds(o, LANES)]
                SC[sl][pl.ds(o, LANES)] = sign / (1.0 + jnp.exp(-mv))
                IX[sl][pl.ds(o, LANES)] = PK[sl][0, pl.ds(o, LANES)]
                DX[sl][pl.ds(o, LANES)] = PK[sl][1, pl.ds(o, LANES)]
                return 0

            lax.fori_loop(0, CH // LANES, _vec, 0)
            pltpu.async_copy(h_hbm.at[IX[sl]], RW[sl], G[sl])

        def _proc(sl):
            pltpu.make_async_copy(h_hbm.at[IX[sl]], RW[sl], G[sl]).wait()

            def _scale(k, _):
                o = pl.multiple_of(k * LANES, 8)
                sv = SC[sl][pl.ds(o, LANES)]
                for i2 in range(LANES):
                    sc = sv[i2]
                    for j in range(DH // LANES):
                        RW[sl][o + i2, pl.ds(j * LANES, LANES)] = (
                            RW[sl][o + i2, pl.ds(j * LANES, LANES)] * sc)
                return 0

            lax.fori_loop(0, CH // LANES, _scale, 0)
            pltpu.async_copy(RW[sl], acc_sh.at[DX[sl]], X[sl], add=True)

        _prep(0, 0)

        def _body(u, _):
            for half in range(2):
                i = 2 * u + half

                @pl.when(s + (i + 1) * NS < NCHT)
                def _():
                    _prep(i + 1, 1 - half)

                _proc(half)
            return 0

        lax.fori_loop(0, (MAXCH - 1) // 2, _body, 0)

        @pl.when(s + (MAXCH - 1) * NS < NCHT)
        def _():
            _proc((MAXCH - 1) % 2)

        for sl in range(2):
            pltpu.make_async_copy(RW[sl], acc_sh.at[DX[sl]],
                                  X[sl]).wait()

    plsc.subcore_barrier()

    for q in range(ROWS_MAIN // CH):
        pltpu.sync_copy(acc_sh.at[pl.ds(base_r + q * CH, CH)],
                        diff_hbm.at[pl.ds(cN + base_r + q * CH, CH)])
    pltpu.sync_copy(
        acc_sh.at[pl.ds(base_r + (ROWS_MAIN // CH) * CH, _rem)],
        diff_hbm.at[pl.ds(cN + base_r + (ROWS_MAIN // CH) * CH, _rem)])

    @pl.when(s < 2)
    def _():
        eb = pl.multiple_of(NS * ROWS_MAIN + s * ROWS_EXTRA, 8)
        pltpu.sync_copy(acc_sh.at[pl.ds(eb, ROWS_EXTRA)],
                        diff_hbm.at[pl.ds(cN + eb, ROWS_EXTRA)])


@functools.partial(
    pl.kernel,
    out_type=jax.ShapeDtypeStruct((NC * 2 * E,), _F32),
    mesh=_mesh,
    scratch_types=(
        [pltpu.VMEM((2, CH), jnp.int32) for _ in range(2)]
        + [pltpu.VMEM((CH,), jnp.int32) for _ in range(2)]
        + [pltpu.VMEM((CH,), jnp.int32) for _ in range(2)]
        + [pltpu.VMEM((CH,), _F32)]
        + [pltpu.VMEM((CH, DH), _F32) for _ in range(2)]
        + [pltpu.VMEM((CH, DH), _F32) for _ in range(2)]
        + [pltpu.SemaphoreType.DMA] * 2
    ),
)
def _bwd(h_hbm, diff_hbm, bpk_hbm,
         dots_hbm, pk0, pk1, ix0, ix1, dx0, dx1, dots_v, hr0, hr1,
         dr0, dr1, g0, g1):
    PK = (pk0, pk1)
    IX = (ix0, ix1)
    DX = (dx0, dx1)
    HR = (hr0, hr1)
    DR = (dr0, dr1)
    G = (g0, g1)
    c = lax.axis_index("c")
    s = lax.axis_index("s")
    cN = c * N
    lane = lax.iota(jnp.int32, LANES)

    for g in range(2):
        def _prep(i, sl, g=g):
            ci = s + i * NS
            pltpu.sync_copy(bpk_hbm.at[c, g, ci], PK[sl])

            def _vec(k, _):
                o = pl.multiple_of(k * LANES, 8)
                IX[sl][pl.ds(o, LANES)] = PK[sl][0, pl.ds(o, LANES)]
                DX[sl][pl.ds(o, LANES)] = PK[sl][1, pl.ds(o, LANES)]
                return 0

            lax.fori_loop(0, CH // LANES, _vec, 0)
            pltpu.async_copy(h_hbm.at[IX[sl]], HR[sl], G[sl])
            pltpu.async_copy(diff_hbm.at[DX[sl]], DR[sl], G[sl])

        def _proc(i, sl, g=g):
            pltpu.make_async_copy(h_hbm.at[IX[sl]], HR[sl], G[sl]).wait()
            pltpu.make_async_copy(diff_hbm.at[DX[sl]], DR[sl],
                                  G[sl]).wait()

            def _dot(k, _):
                o = pl.multiple_of(k * LANES, 8)
                tot = jnp.zeros((LANES,), _F32)
                for i2 in range(LANES):
                    e = o + i2
                    acc = (HR[sl][e, pl.ds(0, LANES)] *
                           DR[sl][e, pl.ds(0, LANES)])
                    for j in range(1, DH // LANES):
                        acc = acc + (
                            HR[sl][e, pl.ds(j * LANES, LANES)] *
                            DR[sl][e, pl.ds(j * LANES, LANES)])
                    for sh in (1, 2, 4, 8):
                        acc = acc + _take16(acc, lane ^ sh)
                    tot = jnp.where(lane == i2, acc, tot)
                dots_v[pl.ds(o, LANES)] = tot
                return 0

            lax.fori_loop(0, CH // LANES, _dot, 0)
            ci = s + i * NS
            fb = pl.multiple_of((c * 2 + g) * E + ci * CH, 8)
            pltpu.sync_copy(dots_v, dots_hbm.at[pl.ds(fb, CH)])

        _prep(0, 0)

        def _body(u, _):
            for half in range(2):
                i = 2 * u + half

                @pl.when(s + (i + 1) * NS < NCHT)
                def _():
                    _prep(i + 1, 1 - half)

                _proc(i, half)
            return 0

        lax.fori_loop(0, (MAXCH - 1) // 2, _body, 0)

        @pl.when(s + (MAXCH - 1) * NS < NCHT)
        def _():
            _proc(MAXCH - 1, (MAXCH - 1) % 2)


def _adam_body(t, diff_ref, dots_ref, em_ref, pm_ref, mem_ref, mpm_ref,
               vem_ref, vpm_ref, oem_ref, opm_ref, omem_ref, ompm_ref,
               ovem_ref, ovpm_ref):
    inv_l = lax.rsqrt(jnp.sum(diff_ref[...] * diff_ref[...]))
    em = em_ref[...]
    pm = pm_ref[...]
    se = jax.nn.sigmoid(em)
    sp = jax.nn.sigmoid(pm)
    d_em = dots_ref[0:ER] + dots_ref[2 * ER:3 * ER]
    d_pm = dots_ref[ER:2 * ER] + dots_ref[3 * ER:4 * ER]
    g_em = se * (1.0 - se) * d_em * inv_l
    g_pm = -sp * (1.0 - sp) * d_pm * inv_l
    bc1 = 1.0 - B1 ** t
    bc2 = 1.0 - B2 ** t
    m_em = B1 * mem_ref[...] + (1.0 - B1) * g_em
    m_pm = B1 * mpm_ref[...] + (1.0 - B1) * g_pm
    v_em = B2 * vem_ref[...] + (1.0 - B2) * g_em * g_em
    v_pm = B2 * vpm_ref[...] + (1.0 - B2) * g_pm * g_pm
    oem_ref[...] = em - LR * (m_em / bc1) / (jnp.sqrt(v_em / bc2) + EPS)
    opm_ref[...] = pm - LR * (m_pm / bc1) / (jnp.sqrt(v_pm / bc2) + EPS)
    omem_ref[...] = m_em
    ompm_ref[...] = m_pm
    ovem_ref[...] = v_em
    ovpm_ref[...] = v_pm


_adam_calls = [
    pl.pallas_call(
        functools.partial(_adam_body, t),
        out_shape=[jax.ShapeDtypeStruct((ER, DH), _F32)] * 6,
    )
    for t in range(1, EPOCHS + 1)
]


def kernel(x, edge_index, perturbed_edge_index, W):
    std = sqrt(2.0) * sqrt(2.0 / (2.0 * N))
    ka, kb = jax.random.split(jax.random.key(1))
    em = jax.random.normal(ka, (E,), _F32) * std
    pm = jax.random.normal(kb, (E,), _F32) * std

    src, dst = edge_index[0], edge_index[1]
    srcp, dstp = perturbed_edge_index[0], perturbed_edge_index[1]

    coff = (jnp.arange(NC, dtype=jnp.int32) * N)[:, None, None, None]
    srcs = jnp.stack([src, srcp]).reshape(2, NCHT, CH)[None]
    dsts = jnp.stack([dst, dstp]).reshape(2, NCHT, CH)[None]
    g_h = srcs + coff
    fpk = jnp.stack([g_h, jnp.broadcast_to(dsts, g_h.shape)], axis=3)
    bpk = jnp.stack([g_h, dsts + coff], axis=3)

    h = _matmul(x, W)

    m_em = jnp.zeros((ER, DH), _F32)
    m_pm = jnp.zeros((ER, DH), _F32)
    v_em = jnp.zeros((ER, DH), _F32)
    v_pm = jnp.zeros((ER, DH), _F32)
    for t in range(1, EPOCHS + 1):
        diff = _fwd(h, fpk, em, pm)
        dots = _bwd(h, diff, bpk)
        em2, pm2, m_em, m_pm, v_em, v_pm = _adam_calls[t - 1](
            diff, dots.reshape(NC * 2 * ER, DH),
            em.reshape(ER, DH), pm.reshape(ER, DH),
            m_em, m_pm, v_em, v_pm)
        em = em2.reshape(E)
        pm = pm2.reshape(E)
    return em, pm
